# baseline (speedup 1.0000x reference)
import functools

import jax
import jax.numpy as jnp
from jax import lax
from jax.experimental import pallas as pl
from jax.experimental.pallas import tpu as pltpu
from jax.experimental.pallas import tpu_sc as plsc

N_ORB = 37
EMBED = 32
W_ROWS = 16
W_FLAT = N_ORB * W_ROWS * EMBED
ELEC_PAD = 3600
NZROWS = 97
N_NODES = 50000

_info = plsc.get_sparse_core_info()
NC = _info.num_cores
NS = _info.num_subcores
NW = NC * NS

NFULL = N_NODES // 128
NCHUNKS = NFULL + 1
K_MAX = (NCHUNKS + NW - 1) // NW


ESTRIDE = EMBED + 1
OSTRIDE = W_ROWS * ESTRIDE
W2_LEN = N_ORB * OSTRIDE


def _sc_body(z_hbm, elec_hbm, w_hbm, out_hbm,
             w_v, w2_v, e_v, zall_v, slota_v, slotb_v, slotc_v, slotd_v,
             sema, semb, semc, semd, zsem):
    c = lax.axis_index("c")
    s = lax.axis_index("s")
    wid = s * NC + c

    for k in range(K_MAX):
        cidx = wid + NW * k

        @pl.when(cidx < NFULL)
        def _():
            pltpu.async_copy(z_hbm.at[pl.ds(cidx * 128, 128)],
                             zall_v.at[pl.ds(k * 128, 128)], zsem)

        @pl.when(cidx == NFULL)
        def _():
            pltpu.async_copy(z_hbm.at[pl.ds(N_NODES - 80, 80)],
                             zall_v.at[pl.ds(k * 128, 80)], zsem)

    pltpu.sync_copy(w_hbm, w_v)
    pltpu.sync_copy(elec_hbm, e_v)
    lanes = lax.iota(jnp.int32, 16)

    for k in range(K_MAX):
        cidx = wid + NW * k

        @pl.when(cidx < NFULL)
        def _():
            pltpu.make_async_copy(z_hbm.at[pl.ds(0, 128)],
                                  zall_v.at[pl.ds(0, 128)], zsem).wait()

        @pl.when(cidx == NFULL)
        def _():
            pltpu.make_async_copy(z_hbm.at[pl.ds(0, 80)],
                                  zall_v.at[pl.ds(0, 80)], zsem).wait()
            zpad = jnp.zeros((16,), jnp.int32)
            zall_v[pl.ds(k * 128 + 80, 16)] = zpad
            zall_v[pl.ds(k * 128 + 96, 16)] = zpad
            zall_v[pl.ds(k * 128 + 112, 16)] = zpad

    def build_w2(k, carry):
        p = k * 16 + lanes
        o = p // OSTRIDE
        r = p - o * OSTRIDE
        e = r // ESTRIDE
        d = r - e * ESTRIDE
        src = o * (W_ROWS * EMBED) + e * EMBED + jnp.minimum(d, EMBED - 1)
        val = plsc.load_gather(w_v, [src])
        val = jnp.where(e == 0, 0.0, val)
        w2_v[pl.ds(k * 16, 16)] = val
        return carry

    lax.fori_loop(0, W2_LEN // 16, build_w2, 0)

    def fill_slot(slot, o, zidx):
        ob = o * OSTRIDE
        for si in range(8):
            e = plsc.load_gather(e_v, [zidx[si] + o])
            we = e * ESTRIDE + ob
            vals = [plsc.load_gather(w2_v, [we + d]) for d in range(EMBED)]
            for d in range(EMBED):
                slot[d, pl.ds(si * 16, 16)] = vals[d]

    def fire(slot, o, n0, sem):
        pltpu.async_copy(slot, out_hbm.at[o, :, pl.ds(n0, 128)], sem)

    def wait(slot, sem):
        pltpu.make_async_copy(slot, out_hbm.at[0, :, pl.ds(0, 128)], sem).wait()

    def do_k(k, carry):
        cidx = wid + NW * k

        @pl.when(cidx < NCHUNKS)
        def _():
            n0 = jnp.where(cidx < NFULL, cidx * 128, N_NODES - 80)
            zidx = [zall_v[pl.ds(k * 128 + si * 16, 16)] * N_ORB
                    for si in range(8)]

            ring = ((slota_v, sema), (slotb_v, semb),
                    (slotc_v, semc), (slotd_v, semd))

            def do_o4(i, carry2):
                not_first = jnp.logical_not((k == 0) & (i == 0))
                for j, (slot, sem) in enumerate(ring):
                    o = 4 * i + j

                    @pl.when(not_first)
                    def _():
                        wait(slot, sem)
                    fill_slot(slot, o, zidx)
                    fire(slot, o, n0, sem)
                return carry2

            lax.fori_loop(0, (N_ORB - 1) // 4, do_o4, 0)

            wait(slota_v, sema)
            fill_slot(slota_v, N_ORB - 1, zidx)
            fire(slota_v, N_ORB - 1, n0, sema)

        return carry

    lax.fori_loop(0, K_MAX, do_k, 0)
    for slot, sem in ((slota_v, sema), (slotb_v, semb),
                      (slotc_v, semc), (slotd_v, semd)):
        wait(slot, sem)


@jax.jit
def _run(z, elec_flat, w_flat):
    mesh = plsc.VectorSubcoreMesh(core_axis_name="c", subcore_axis_name="s")
    f = pl.kernel(
        _sc_body,
        out_type=jax.ShapeDtypeStruct((N_ORB, EMBED, N_NODES), jnp.float32),
        mesh=mesh,
        compiler_params=pltpu.CompilerParams(
            needs_layout_passes=False, use_tc_tiling_on_sc=True),
        scratch_types=[
            pltpu.VMEM((W_FLAT,), jnp.float32),
            pltpu.VMEM((W2_LEN,), jnp.float32),
            pltpu.VMEM((ELEC_PAD,), jnp.int32),
            pltpu.VMEM((K_MAX * 128,), jnp.int32),
            pltpu.VMEM((EMBED, 128), jnp.float32),
            pltpu.VMEM((EMBED, 128), jnp.float32),
            pltpu.VMEM((EMBED, 128), jnp.float32),
            pltpu.VMEM((EMBED, 128), jnp.float32),
            pltpu.SemaphoreType.DMA,
            pltpu.SemaphoreType.DMA,
            pltpu.SemaphoreType.DMA,
            pltpu.SemaphoreType.DMA,
            pltpu.SemaphoreType.DMA,
        ],
    )
    return f(z, elec_flat, w_flat)


def kernel(z, elec_table, W):
    elec_flat = jnp.zeros((ELEC_PAD,), jnp.int32).at[: NZROWS * N_ORB].set(
        elec_table.reshape(-1))
    y = _run(z, elec_flat, W.reshape(-1))
    return jnp.transpose(y, (2, 0, 1))


# speedup vs baseline: 1.0122x; 1.0122x over previous
import functools

import jax
import jax.numpy as jnp
from jax import lax
from jax.experimental import pallas as pl
from jax.experimental.pallas import tpu as pltpu
from jax.experimental.pallas import tpu_sc as plsc

N_ORB = 37
EMBED = 32
W_ROWS = 16
W_FLAT = N_ORB * W_ROWS * EMBED
ELEC_PAD = 3600
NZROWS = 97
N_NODES = 50000

_info = plsc.get_sparse_core_info()
NC = _info.num_cores
NS = _info.num_subcores
NW = NC * NS

NFULL = N_NODES // 128
NCHUNKS = NFULL + 1
K_MAX = (NCHUNKS + NW - 1) // NW


ESTRIDE = EMBED + 1
OSTRIDE = W_ROWS * ESTRIDE
W2_LEN = N_ORB * OSTRIDE


def _sc_body(z_hbm, elec_hbm, w_hbm, out_hbm,
             w_v, w2_v, e_v, zall_v, slota_v, slotb_v, slote_v,
             sema, semb, seme, zsem):
    c = lax.axis_index("c")
    s = lax.axis_index("s")
    wid = s * NC + c

    for k in range(K_MAX):
        cidx = wid + NW * k

        @pl.when(cidx < NFULL)
        def _():
            pltpu.async_copy(z_hbm.at[pl.ds(cidx * 128, 128)],
                             zall_v.at[pl.ds(k * 128, 128)], zsem)

        @pl.when(cidx == NFULL)
        def _():
            pltpu.async_copy(z_hbm.at[pl.ds(N_NODES - 80, 80)],
                             zall_v.at[pl.ds(k * 128, 80)], zsem)

    pltpu.sync_copy(w_hbm, w_v)
    pltpu.sync_copy(elec_hbm, e_v)
    lanes = lax.iota(jnp.int32, 16)

    for k in range(K_MAX):
        cidx = wid + NW * k

        @pl.when(cidx < NFULL)
        def _():
            pltpu.make_async_copy(z_hbm.at[pl.ds(0, 128)],
                                  zall_v.at[pl.ds(0, 128)], zsem).wait()

        @pl.when(cidx == NFULL)
        def _():
            pltpu.make_async_copy(z_hbm.at[pl.ds(0, 80)],
                                  zall_v.at[pl.ds(0, 80)], zsem).wait()
            zpad = jnp.zeros((16,), jnp.int32)
            zall_v[pl.ds(k * 128 + 80, 16)] = zpad
            zall_v[pl.ds(k * 128 + 96, 16)] = zpad
            zall_v[pl.ds(k * 128 + 112, 16)] = zpad

    def build_w2(k, carry):
        p = k * 16 + lanes
        o = p // OSTRIDE
        r = p - o * OSTRIDE
        e = r // ESTRIDE
        d = r - e * ESTRIDE
        src = o * (W_ROWS * EMBED) + e * EMBED + jnp.minimum(d, EMBED - 1)
        val = plsc.load_gather(w_v, [src])
        val = jnp.where(e == 0, 0.0, val)
        w2_v[pl.ds(k * 16, 16)] = val
        return carry

    lax.fori_loop(0, W2_LEN // 16, build_w2, 0)

    def fill_rows(slot, obase, nors, zidx):
        for jo in range(nors):
            o = obase + jo
            ob = o * OSTRIDE
            for si in range(8):
                e = plsc.load_gather(e_v, [zidx[si] + o])
                we = e * ESTRIDE + ob
                vals = [plsc.load_gather(w2_v, [we + d]) for d in range(EMBED)]
                for d in range(EMBED):
                    slot[jo * EMBED + d, pl.ds(si * 16, 16)] = vals[d]

    def fire(slot, obase, nors, n0, sem):
        pltpu.async_copy(
            slot, out_hbm.at[pl.ds(obase * EMBED, nors * EMBED),
                             pl.ds(n0, 128)], sem)

    def wait(nors, sem):
        pltpu.make_async_copy(
            slota_v.at[pl.ds(0, nors * EMBED)],
            out_hbm.at[pl.ds(0, nors * EMBED), pl.ds(0, 128)], sem).wait()

    def do_k(k, carry):
        cidx = wid + NW * k

        @pl.when(cidx < NCHUNKS)
        def _():
            n0 = jnp.where(cidx < NFULL, cidx * 128, N_NODES - 80)
            zidx = [zall_v[pl.ds(k * 128 + si * 16, 16)] * N_ORB
                    for si in range(8)]

            def do_q(q, carry2):
                not_first = jnp.logical_not((k == 0) & (q == 0))

                @pl.when(not_first)
                def _():
                    wait(2, sema)
                fill_rows(slota_v, 4 * q, 2, zidx)
                fire(slota_v, 4 * q, 2, n0, sema)

                @pl.when(not_first)
                def _():
                    wait(2, semb)
                fill_rows(slotb_v, 4 * q + 2, 2, zidx)
                fire(slotb_v, 4 * q + 2, 2, n0, semb)
                return carry2

            lax.fori_loop(0, N_ORB // 4, do_q, 0)

            @pl.when(k > 0)
            def _():
                wait(1, seme)
            fill_rows(slote_v, N_ORB - 1, 1, zidx)
            fire(slote_v, N_ORB - 1, 1, n0, seme)

        return carry

    lax.fori_loop(0, K_MAX, do_k, 0)
    wait(2, sema)
    wait(2, semb)
    wait(1, seme)


@jax.jit
def _run(z, elec_flat, w_flat):
    mesh = plsc.VectorSubcoreMesh(core_axis_name="c", subcore_axis_name="s")
    f = pl.kernel(
        _sc_body,
        out_type=jax.ShapeDtypeStruct((N_ORB * EMBED, N_NODES), jnp.float32),
        mesh=mesh,
        compiler_params=pltpu.CompilerParams(
            needs_layout_passes=False, use_tc_tiling_on_sc=True),
        scratch_types=[
            pltpu.VMEM((W_FLAT,), jnp.float32),
            pltpu.VMEM((W2_LEN,), jnp.float32),
            pltpu.VMEM((ELEC_PAD,), jnp.int32),
            pltpu.VMEM((K_MAX * 128,), jnp.int32),
            pltpu.VMEM((2 * EMBED, 128), jnp.float32),
            pltpu.VMEM((2 * EMBED, 128), jnp.float32),
            pltpu.VMEM((EMBED, 128), jnp.float32),
            pltpu.SemaphoreType.DMA,
            pltpu.SemaphoreType.DMA,
            pltpu.SemaphoreType.DMA,
            pltpu.SemaphoreType.DMA,
        ],
    )
    return f(z, elec_flat, w_flat)


def kernel(z, elec_table, W):
    elec_flat = jnp.zeros((ELEC_PAD,), jnp.int32).at[: NZROWS * N_ORB].set(
        elec_table.reshape(-1))
    y = _run(z, elec_flat, W.reshape(-1))
    y = y.reshape(N_ORB, EMBED, N_NODES)
    return jnp.transpose(y, (2, 0, 1))


# speedup vs baseline: 2.0113x; 1.9870x over previous
import functools

import jax
import jax.numpy as jnp
from jax import lax
from jax.experimental import pallas as pl
from jax.experimental.pallas import tpu as pltpu
from jax.experimental.pallas import tpu_sc as plsc

N_ORB = 37
EMBED = 32
W_ROWS = 16
W_FLAT = N_ORB * W_ROWS * EMBED
ELEC_PAD = 3600
NZROWS = 97
N_NODES = 50000

_info = plsc.get_sparse_core_info()
NC = _info.num_cores
NS = _info.num_subcores
NW = NC * NS

NFULL = N_NODES // 128
NCHUNKS = NFULL + 1
K_MAX = (NCHUNKS + NW - 1) // NW


ESTRIDE = EMBED + 1
OSTRIDE = W_ROWS * ESTRIDE
W2_LEN = N_ORB * OSTRIDE


def _sc_body(z_hbm, elec_hbm, w_hbm, out_hbm,
             w_v, w2_v, e_v, zall_v, zi_v, slota_v, slotb_v, slotc_v, slote_v,
             sema, semb, semc, seme, zsem):
    c = lax.axis_index("c")
    s = lax.axis_index("s")
    wid = s * NC + c

    for k in range(K_MAX):
        cidx = wid + NW * k

        @pl.when(cidx < NFULL)
        def _():
            pltpu.async_copy(z_hbm.at[pl.ds(cidx * 128, 128)],
                             zall_v.at[pl.ds(k * 128, 128)], zsem)

        @pl.when(cidx == NFULL)
        def _():
            pltpu.async_copy(z_hbm.at[pl.ds(N_NODES - 80, 80)],
                             zall_v.at[pl.ds(k * 128, 80)], zsem)

    pltpu.sync_copy(w_hbm, w_v)
    pltpu.sync_copy(elec_hbm, e_v)
    lanes = lax.iota(jnp.int32, 16)

    for k in range(K_MAX):
        cidx = wid + NW * k

        @pl.when(cidx < NFULL)
        def _():
            pltpu.make_async_copy(z_hbm.at[pl.ds(0, 128)],
                                  zall_v.at[pl.ds(0, 128)], zsem).wait()

        @pl.when(cidx == NFULL)
        def _():
            pltpu.make_async_copy(z_hbm.at[pl.ds(0, 80)],
                                  zall_v.at[pl.ds(0, 80)], zsem).wait()
            zpad = jnp.zeros((16,), jnp.int32)
            zall_v[pl.ds(k * 128 + 80, 16)] = zpad
            zall_v[pl.ds(k * 128 + 96, 16)] = zpad
            zall_v[pl.ds(k * 128 + 112, 16)] = zpad

    def build_w2(k, carry):
        p = k * 16 + lanes
        o = p // OSTRIDE
        r = p - o * OSTRIDE
        e = r // ESTRIDE
        d = r - e * ESTRIDE
        src = o * (W_ROWS * EMBED) + e * EMBED + jnp.minimum(d, EMBED - 1)
        val = plsc.load_gather(w_v, [src])
        val = jnp.where(e == 0, 0.0, val)
        w2_v[pl.ds(k * 16, 16)] = val
        return carry

    lax.fori_loop(0, W2_LEN // 16, build_w2, 0)

    def fill_rows(slot, obase, nors):
        for jo in range(nors):
            o = obase + jo
            ob = o * OSTRIDE

            def do_si(si, carry3):
                zi = zi_v[pl.ds(si * 16, 16)]
                e = plsc.load_gather(e_v, [zi + o])
                we = e * ESTRIDE + ob
                vals = [plsc.load_gather(w2_v, [we + d]) for d in range(EMBED)]
                for d in range(EMBED):
                    slot[jo * EMBED + d, pl.ds(si * 16, 16)] = vals[d]
                return carry3

            lax.fori_loop(0, 8, do_si, 0)

    def fire(slot, obase, nors, n0, sem):
        pltpu.async_copy(
            slot, out_hbm.at[pl.ds(obase * EMBED, nors * EMBED),
                             pl.ds(n0, 128)], sem)

    def wait(nors, sem):
        pltpu.make_async_copy(
            slota_v.at[pl.ds(0, nors * EMBED)],
            out_hbm.at[pl.ds(0, nors * EMBED), pl.ds(0, 128)], sem).wait()

    def do_k(k, carry):
        cidx = wid + NW * k

        @pl.when(cidx < NCHUNKS)
        def _():
            n0 = jnp.where(cidx < NFULL, cidx * 128, N_NODES - 80)
            for si in range(8):
                zi_v[pl.ds(si * 16, 16)] = (
                    zall_v[pl.ds(k * 128 + si * 16, 16)] * N_ORB)

            def do_r(r, carry2):
                not_first = jnp.logical_not((k == 0) & (r == 0))
                for j, (slot, sem) in enumerate(
                        ((slota_v, sema), (slotb_v, semb), (slotc_v, semc))):
                    @pl.when(not_first)
                    def _():
                        wait(3, sem)
                    fill_rows(slot, 9 * r + 3 * j, 3)
                    fire(slot, 9 * r + 3 * j, 3, n0, sem)
                return carry2

            lax.fori_loop(0, 4, do_r, 0)

            @pl.when(k > 0)
            def _():
                wait(1, seme)
            fill_rows(slote_v, N_ORB - 1, 1)
            fire(slote_v, N_ORB - 1, 1, n0, seme)

        return carry

    lax.fori_loop(0, K_MAX, do_k, 0)
    wait(3, sema)
    wait(3, semb)
    wait(3, semc)
    wait(1, seme)


@jax.jit
def _run(z, elec_flat, w_flat):
    mesh = plsc.VectorSubcoreMesh(core_axis_name="c", subcore_axis_name="s")
    f = pl.kernel(
        _sc_body,
        out_type=jax.ShapeDtypeStruct((N_ORB * EMBED, N_NODES), jnp.float32),
        mesh=mesh,
        compiler_params=pltpu.CompilerParams(
            needs_layout_passes=False, use_tc_tiling_on_sc=True),
        scratch_types=[
            pltpu.VMEM((W_FLAT,), jnp.float32),
            pltpu.VMEM((W2_LEN,), jnp.float32),
            pltpu.VMEM((ELEC_PAD,), jnp.int32),
            pltpu.VMEM((K_MAX * 128,), jnp.int32),
            pltpu.VMEM((128,), jnp.int32),
            pltpu.VMEM((3 * EMBED, 128), jnp.float32),
            pltpu.VMEM((3 * EMBED, 128), jnp.float32),
            pltpu.VMEM((3 * EMBED, 128), jnp.float32),
            pltpu.VMEM((EMBED, 128), jnp.float32),
            pltpu.SemaphoreType.DMA,
            pltpu.SemaphoreType.DMA,
            pltpu.SemaphoreType.DMA,
            pltpu.SemaphoreType.DMA,
            pltpu.SemaphoreType.DMA,
        ],
    )
    return f(z, elec_flat, w_flat)


def kernel(z, elec_table, W):
    elec_flat = jnp.zeros((ELEC_PAD,), jnp.int32).at[: NZROWS * N_ORB].set(
        elec_table.reshape(-1))
    y = _run(z, elec_flat, W.reshape(-1))
    y = y.reshape(N_ORB, EMBED, N_NODES)
    return jnp.transpose(y, (2, 0, 1))


# speedup vs baseline: 2.0157x; 1.0022x over previous
import jax
import jax.numpy as jnp
from jax import lax
from jax.experimental import pallas as pl
from jax.experimental.pallas import tpu as pltpu
from jax.experimental.pallas import tpu_sc as plsc

N_ORB = 37
EMBED = 32
W_ROWS = 16
W_FLAT = N_ORB * W_ROWS * EMBED
ELEC_PAD = 3600
NZROWS = 97
N_NODES = 50000

_info = plsc.get_sparse_core_info()
NC = _info.num_cores
NS = _info.num_subcores
NW = NC * NS

NFULL = N_NODES // 128
NCHUNKS = NFULL + 1
K_MAX = (NCHUNKS + NW - 1) // NW


ESTRIDE = EMBED + 1
OSTRIDE = W_ROWS * ESTRIDE
W2_LEN = N_ORB * OSTRIDE


def _sc_body(z_hbm, elec_hbm, w_hbm, out_hbm,
             w_v, w2_v, e_v, zall_v, zi_v, slota_v, slotb_v, slotc_v, slote_v,
             sema, semb, semc, seme, zsem):
    c = lax.axis_index("c")
    s = lax.axis_index("s")
    wid = s * NC + c

    for k in range(K_MAX):
        cidx = wid + NW * k

        @pl.when(cidx < NFULL)
        def _():
            pltpu.async_copy(z_hbm.at[pl.ds(cidx * 128, 128)],
                             zall_v.at[pl.ds(k * 128, 128)], zsem)

        @pl.when(cidx == NFULL)
        def _():
            pltpu.async_copy(z_hbm.at[pl.ds(N_NODES - 80, 80)],
                             zall_v.at[pl.ds(k * 128, 80)], zsem)

    pltpu.sync_copy(w_hbm, w_v)
    pltpu.sync_copy(elec_hbm, e_v)
    lanes = lax.iota(jnp.int32, 16)

    for k in range(K_MAX):
        cidx = wid + NW * k

        @pl.when(cidx < NFULL)
        def _():
            pltpu.make_async_copy(z_hbm.at[pl.ds(0, 128)],
                                  zall_v.at[pl.ds(0, 128)], zsem).wait()

        @pl.when(cidx == NFULL)
        def _():
            pltpu.make_async_copy(z_hbm.at[pl.ds(0, 80)],
                                  zall_v.at[pl.ds(0, 80)], zsem).wait()
            zpad = jnp.zeros((16,), jnp.int32)
            zall_v[pl.ds(k * 128 + 80, 16)] = zpad
            zall_v[pl.ds(k * 128 + 96, 16)] = zpad
            zall_v[pl.ds(k * 128 + 112, 16)] = zpad

    def build_w2(k, carry):
        p = k * 16 + lanes
        o = p // OSTRIDE
        r = p - o * OSTRIDE
        e = r // ESTRIDE
        d = r - e * ESTRIDE
        src = o * (W_ROWS * EMBED) + e * EMBED + jnp.minimum(d, EMBED - 1)
        val = plsc.load_gather(w_v, [src])
        val = jnp.where(e == 0, 0.0, val)
        w2_v[pl.ds(k * 16, 16)] = val
        return carry

    lax.fori_loop(0, W2_LEN // 16, build_w2, 0)

    def fill_rows(slot, obase, nors):
        for jo in range(nors):
            o = obase + jo
            ob = o * OSTRIDE

            def do_si(si, carry3):
                zi = zi_v[pl.ds(si * 16, 16)]
                e = plsc.load_gather(e_v, [zi + o])
                we = e * ESTRIDE + ob
                vals = [plsc.load_gather(w2_v, [we + d]) for d in range(EMBED)]
                for d in range(EMBED):
                    slot[jo * EMBED + d, pl.ds(si * 16, 16)] = vals[d]
                return carry3

            lax.fori_loop(0, 8, do_si, 0)

    def fire(slot, obase, nors, n0, sem):
        pltpu.async_copy(
            slot, out_hbm.at[pl.ds(obase * EMBED, nors * EMBED),
                             pl.ds(n0, 128)], sem)

    def wait(nors, sem):
        pltpu.make_async_copy(
            slota_v.at[pl.ds(0, nors * EMBED)],
            out_hbm.at[pl.ds(0, nors * EMBED), pl.ds(0, 128)], sem).wait()

    def do_k(k, carry):
        cidx = wid + NW * k

        @pl.when(cidx < NCHUNKS)
        def _():
            n0 = jnp.where(cidx < NFULL, cidx * 128, N_NODES - 80)
            for si in range(8):
                zi_v[pl.ds(si * 16, 16)] = (
                    zall_v[pl.ds(k * 128 + si * 16, 16)] * N_ORB)

            def do_r(r, carry2):
                not_first = jnp.logical_not((k == 0) & (r == 0))
                for j, (slot, sem) in enumerate(
                        ((slota_v, sema), (slotb_v, semb), (slotc_v, semc))):
                    @pl.when(not_first)
                    def _():
                        wait(3, sem)
                    fill_rows(slot, 9 * r + 3 * j, 3)
                    fire(slot, 9 * r + 3 * j, 3, n0, sem)
                return carry2

            lax.fori_loop(0, 4, do_r, 0)

            @pl.when(k > 0)
            def _():
                wait(1, seme)
            fill_rows(slote_v, N_ORB - 1, 1)
            fire(slote_v, N_ORB - 1, 1, n0, seme)

        return carry

    lax.fori_loop(0, K_MAX, do_k, 0)
    wait(3, sema)
    wait(3, semb)
    wait(3, semc)
    wait(1, seme)


@jax.jit
def _run(z, elec_flat, w_flat):
    mesh = plsc.VectorSubcoreMesh(core_axis_name="c", subcore_axis_name="s")
    f = pl.kernel(
        _sc_body,
        out_type=jax.ShapeDtypeStruct((N_ORB * EMBED, N_NODES), jnp.float32),
        mesh=mesh,
        compiler_params=pltpu.CompilerParams(
            needs_layout_passes=False, use_tc_tiling_on_sc=True),
        scratch_types=[
            pltpu.VMEM((W_FLAT,), jnp.float32),
            pltpu.VMEM((W2_LEN,), jnp.float32),
            pltpu.VMEM((ELEC_PAD,), jnp.int32),
            pltpu.VMEM((K_MAX * 128,), jnp.int32),
            pltpu.VMEM((128,), jnp.int32),
            pltpu.VMEM((3 * EMBED, 128), jnp.float32),
            pltpu.VMEM((3 * EMBED, 128), jnp.float32),
            pltpu.VMEM((3 * EMBED, 128), jnp.float32),
            pltpu.VMEM((EMBED, 128), jnp.float32),
            pltpu.SemaphoreType.DMA,
            pltpu.SemaphoreType.DMA,
            pltpu.SemaphoreType.DMA,
            pltpu.SemaphoreType.DMA,
            pltpu.SemaphoreType.DMA,
        ],
    )
    return f(z, elec_flat, w_flat)


def kernel(z, elec_table, W):
    elec_flat = jnp.zeros((ELEC_PAD,), jnp.int32).at[: NZROWS * N_ORB].set(
        elec_table.reshape(-1))
    y = _run(z, elec_flat, W.reshape(-1))
    y = y.reshape(N_ORB, EMBED, N_NODES)
    return jnp.transpose(y, (2, 0, 1))
